# baseline (speedup 1.0000x reference)
import functools

import jax
import jax.numpy as jnp
from jax import lax
from jax.experimental import pallas as pl
from jax.experimental.pallas import tpu as pltpu
from jax.experimental.pallas import tpu_sc as plsc


BM = 128
MIN_BLKS = 4
DUMMY = 4
QUOTA = 2


def _sc_scatter_rows(x, pos_k, cap_rows):
    s, d = x.shape
    info = plsc.get_sparse_core_info()
    nw = info.num_cores * info.num_subcores
    c = s // nw
    mesh = plsc.VectorSubcoreMesh(core_axis_name="c", subcore_axis_name="s")

    @functools.partial(
        pl.kernel, mesh=mesh,
        out_type=jax.ShapeDtypeStruct((cap_rows, d), jnp.float32),
        scratch_types=[
            pltpu.VMEM((c, d), jnp.float32),
            pltpu.VMEM((c,), jnp.int32),
            pltpu.VMEM((c,), jnp.int32),
            pltpu.SemaphoreType.DMA,
        ],
    )
    def body(x_hbm, pos_hbm, xg_hbm, rows_v, idx0_v, idx1_v, sem):
        wid = lax.axis_index("s") * info.num_cores + lax.axis_index("c")
        base = wid * c
        pltpu.sync_copy(x_hbm.at[pl.ds(base, c)], rows_v)
        pltpu.sync_copy(pos_hbm.at[0, pl.ds(base, c)], idx0_v)
        pltpu.sync_copy(pos_hbm.at[1, pl.ds(base, c)], idx1_v)
        cp0 = pltpu.async_copy(rows_v, xg_hbm.at[idx0_v], sem)
        cp1 = pltpu.async_copy(rows_v, xg_hbm.at[idx1_v], sem)
        cp0.wait()
        cp1.wait()

    return body(x, pos_k)


def _sc_gather_rows(table, idx, chunk):
    b = idx.shape[0]
    d = table.shape[1]
    info = plsc.get_sparse_core_info()
    nw = info.num_cores * info.num_subcores
    b_per_w = b // nw
    n_ch = b_per_w // chunk
    mesh = plsc.VectorSubcoreMesh(core_axis_name="c", subcore_axis_name="s")

    @functools.partial(
        pl.kernel, mesh=mesh,
        out_type=jax.ShapeDtypeStruct((b, d), jnp.float32),
        scratch_types=[
            pltpu.VMEM((chunk, d), jnp.float32),
            pltpu.VMEM((chunk,), jnp.int32),
            pltpu.SemaphoreType.DMA,
        ],
    )
    def body(table_hbm, idx_hbm, out_hbm, rows_v, idx_v, sem):
        wid = lax.axis_index("s") * info.num_cores + lax.axis_index("c")
        base = wid * b_per_w
        for j in range(n_ch):
            off = base + j * chunk
            pltpu.sync_copy(idx_hbm.at[pl.ds(off, chunk)], idx_v)
            pltpu.async_copy(table_hbm.at[idx_v], rows_v, sem).wait()
            pltpu.sync_copy(rows_v, out_hbm.at[pl.ds(off, chunk)])

    return body(table, idx)


def _stage13_body(par_ref, en_ref, cidx_ref, xg_ref, w1_ref, w3_ref, h_ref,
                  w1b_ref, w3b_ref, stg_ref, sem_ref):
    m = pl.program_id(0)
    inter = w1b_ref.shape[1]
    wc = inter // (MIN_BLKS * QUOTA)
    par = par_ref[m]
    en = en_ref[m]

    def mk(j, w_hbm):
        ci = cidx_ref[m, j % QUOTA]
        src = w_hbm.at[en, pl.ds(ci * wc, wc), :]
        return pltpu.make_async_copy(src, stg_ref.at[j], sem_ref.at[j])

    srcs = [w1_ref, w1_ref, w3_ref, w3_ref]
    for j in range(2 * QUOTA):
        mk(j, srcs[j]).start()

    xb = xg_ref[...].astype(jnp.bfloat16)
    dn = (((1,), (1,)), ((), ()))
    x1 = lax.dot_general(xb, w1b_ref[par], dn,
                         preferred_element_type=jnp.float32)
    x3 = lax.dot_general(xb, w3b_ref[par], dn,
                         preferred_element_type=jnp.float32)
    h_ref[...] = (x1 * jax.nn.sigmoid(x1) * x3).astype(jnp.bfloat16)

    dsts = [w1b_ref, w1b_ref, w3b_ref, w3b_ref]
    for j in range(2 * QUOTA):
        ci = cidx_ref[m, j % QUOTA]
        mk(j, srcs[j]).wait()
        dsts[j][1 - par, pl.ds(ci * wc, wc), :] = (
            stg_ref[j].astype(jnp.bfloat16))


def _stage2_body(par_ref, en_ref, cidx_ref, h_ref, w2_ref, out_ref,
                 w2b_ref, stg_ref, sem_ref):
    m = pl.program_id(0)
    d = w2b_ref.shape[1]
    wc = d // (MIN_BLKS * QUOTA)
    par = par_ref[m]
    en = en_ref[m]

    def mk(j):
        ci = cidx_ref[m, j]
        src = w2_ref.at[en, pl.ds(ci * wc, wc), :]
        return pltpu.make_async_copy(src, stg_ref.at[j], sem_ref.at[j])

    for j in range(QUOTA):
        mk(j).start()

    dn = (((1,), (1,)), ((), ()))
    out_ref[...] = lax.dot_general(h_ref[...], w2b_ref[par], dn,
                                   preferred_element_type=jnp.float32)

    for j in range(QUOTA):
        ci = cidx_ref[m, j]
        mk(j).wait()
        w2b_ref[1 - par, pl.ds(ci * wc, wc), :] = (
            stg_ref[j].astype(jnp.bfloat16))


def _grouped_ffn(xg, w1, w3, w2, par, en, cidx, ngrid, cap_rows):
    num_e, inter, dim = w1.shape
    wc13 = inter // (MIN_BLKS * QUOTA)
    wc2 = dim // (MIN_BLKS * QUOTA)

    h = pl.pallas_call(
        _stage13_body,
        grid_spec=pltpu.PrefetchScalarGridSpec(
            num_scalar_prefetch=3,
            grid=(ngrid,),
            in_specs=[
                pl.BlockSpec((BM, dim), lambda m, par, en, ci: (m, 0)),
                pl.BlockSpec(memory_space=pltpu.MemorySpace.HBM),
                pl.BlockSpec(memory_space=pltpu.MemorySpace.HBM),
            ],
            out_specs=pl.BlockSpec((BM, inter), lambda m, par, en, ci: (m, 0)),
            scratch_shapes=[
                pltpu.VMEM((2, inter, dim), jnp.bfloat16),
                pltpu.VMEM((2, inter, dim), jnp.bfloat16),
                pltpu.VMEM((2 * QUOTA, wc13, dim), jnp.float32),
                pltpu.SemaphoreType.DMA((2 * QUOTA,)),
            ],
        ),
        out_shape=jax.ShapeDtypeStruct((cap_rows, inter), jnp.bfloat16),
        compiler_params=pltpu.CompilerParams(
            dimension_semantics=("arbitrary",)),
    )(par, en, cidx, xg, w1, w3)

    y = pl.pallas_call(
        _stage2_body,
        grid_spec=pltpu.PrefetchScalarGridSpec(
            num_scalar_prefetch=3,
            grid=(ngrid,),
            in_specs=[
                pl.BlockSpec((BM, inter), lambda m, par, en, ci: (m, 0)),
                pl.BlockSpec(memory_space=pltpu.MemorySpace.HBM),
            ],
            out_specs=pl.BlockSpec((BM, dim), lambda m, par, en, ci: (m, 0)),
            scratch_shapes=[
                pltpu.VMEM((2, dim, inter), jnp.bfloat16),
                pltpu.VMEM((QUOTA, wc2, inter), jnp.float32),
                pltpu.SemaphoreType.DMA((QUOTA,)),
            ],
        ),
        out_shape=jax.ShapeDtypeStruct((cap_rows, dim), jnp.float32),
        compiler_params=pltpu.CompilerParams(
            dimension_semantics=("arbitrary",)),
    )(par, en, cidx, h, w2)
    return y


def kernel(x, expert_indices, w1, w2, w3):
    seq_len, dim = x.shape
    top_k = expert_indices.shape[1]
    num_e = w1.shape[0]
    p = seq_len * top_k
    m_max = DUMMY + p // BM + (num_e - 1) * MIN_BLKS
    cap_rows = m_max * BM

    e_flat = expert_indices.reshape(-1).astype(jnp.int32)
    oh = (e_flat[:, None] == jnp.arange(num_e, dtype=jnp.int32)[None, :])
    oh = oh.astype(jnp.int32)
    counts = oh.sum(0)
    present = (counts > 0).astype(jnp.int32)
    nb = jnp.maximum((counts + BM - 1) // BM, MIN_BLKS) * present
    starts_blk = DUMMY + jnp.concatenate(
        [jnp.zeros((1,), jnp.int32), jnp.cumsum(nb)[:-1].astype(jnp.int32)])
    rank = (jnp.cumsum(oh, axis=0) * oh).sum(1) - 1
    pos = starts_blk[e_flat] * BM + rank
    ngrid = DUMMY + nb.sum()

    bids = jnp.arange(m_max, dtype=jnp.int32)
    is_dummy = bids < DUMMY
    eb = jnp.clip(
        jnp.searchsorted(starts_blk, bids, side="right").astype(jnp.int32) - 1,
        0, num_e - 1)
    eb = jnp.where(is_dummy, 0, eb)
    runidx = jnp.where(is_dummy, 0, jnp.cumsum(present)[eb])
    par = (runidx % 2).astype(jnp.int32)
    sir = jnp.where(is_dummy, bids, bids - starts_blk[eb])
    big = jnp.where(present > 0, jnp.arange(num_e, dtype=jnp.int32), num_e)
    sufmin = jnp.flip(lax.cummin(jnp.flip(big)))
    nxt = jnp.concatenate([sufmin[1:],
                           jnp.full((1,), num_e, jnp.int32)]).astype(jnp.int32)
    en = jnp.where(is_dummy, sufmin[0], nxt[eb])
    en = jnp.where(en >= num_e, eb, en).astype(jnp.int32)
    cidx = jnp.minimum(sir[:, None] * QUOTA
                       + jnp.arange(QUOTA, dtype=jnp.int32)[None, :],
                       MIN_BLKS * QUOTA - 1).astype(jnp.int32)
    pos_k = pos.reshape(seq_len, top_k).T

    xg = _sc_scatter_rows(x, pos_k, cap_rows)

    y = _grouped_ffn(xg, w1, w3, w2, par, en, cidx, ngrid, cap_rows)

    out = _sc_gather_rows(y, pos, 64)
    return out.reshape(seq_len, top_k, dim)


# speedup vs baseline: 1.2459x; 1.2459x over previous
import functools

import jax
import jax.numpy as jnp
from jax import lax
from jax.experimental import pallas as pl
from jax.experimental.pallas import tpu as pltpu
from jax.experimental.pallas import tpu_sc as plsc


BM = 128
MIN_BLKS = 5
DUMMY = 5
QUOTA = 2
NCH = 8


def _sc_scatter_rows(x, pos_k, cap_rows):
    s, d = x.shape
    info = plsc.get_sparse_core_info()
    nw = info.num_cores * info.num_subcores
    c = s // nw
    mesh = plsc.VectorSubcoreMesh(core_axis_name="c", subcore_axis_name="s")

    @functools.partial(
        pl.kernel, mesh=mesh,
        out_type=jax.ShapeDtypeStruct((cap_rows, d), jnp.float32),
        scratch_types=[
            pltpu.VMEM((c, d), jnp.float32),
            pltpu.VMEM((c,), jnp.int32),
            pltpu.VMEM((c,), jnp.int32),
            pltpu.SemaphoreType.DMA,
        ],
    )
    def body(x_hbm, pos_hbm, xg_hbm, rows_v, idx0_v, idx1_v, sem):
        wid = lax.axis_index("s") * info.num_cores + lax.axis_index("c")
        base = wid * c
        pltpu.sync_copy(x_hbm.at[pl.ds(base, c)], rows_v)
        pltpu.sync_copy(pos_hbm.at[0, pl.ds(base, c)], idx0_v)
        pltpu.sync_copy(pos_hbm.at[1, pl.ds(base, c)], idx1_v)
        cp0 = pltpu.async_copy(rows_v, xg_hbm.at[idx0_v], sem)
        cp1 = pltpu.async_copy(rows_v, xg_hbm.at[idx1_v], sem)
        cp0.wait()
        cp1.wait()

    return body(x, pos_k)


def _sc_gather_rows(table, idx, chunk):
    b = idx.shape[0]
    d = table.shape[1]
    info = plsc.get_sparse_core_info()
    nw = info.num_cores * info.num_subcores
    b_per_w = b // nw
    n_ch = b_per_w // chunk
    mesh = plsc.VectorSubcoreMesh(core_axis_name="c", subcore_axis_name="s")

    @functools.partial(
        pl.kernel, mesh=mesh,
        out_type=jax.ShapeDtypeStruct((b, d), jnp.float32),
        scratch_types=[
            pltpu.VMEM((chunk, d), jnp.float32),
            pltpu.VMEM((chunk,), jnp.int32),
            pltpu.SemaphoreType.DMA,
        ],
    )
    def body(table_hbm, idx_hbm, out_hbm, rows_v, idx_v, sem):
        wid = lax.axis_index("s") * info.num_cores + lax.axis_index("c")
        base = wid * b_per_w
        for j in range(n_ch):
            off = base + j * chunk
            pltpu.sync_copy(idx_hbm.at[pl.ds(off, chunk)], idx_v)
            pltpu.async_copy(table_hbm.at[idx_v], rows_v, sem).wait()
            pltpu.sync_copy(rows_v, out_hbm.at[pl.ds(off, chunk)])

    return body(table, idx)


def _stage13_body(par_ref, en_ref, fidx_ref, cen_ref, cidx_ref, cslab_ref,
                  xg_ref, w1_ref, w3_ref, h_ref,
                  w1b_ref, w3b_ref, stg_ref, sem_ref):
    m = pl.program_id(0)
    inter = w1b_ref.shape[1]
    wc = inter // NCH
    par = par_ref[m]
    nslots = 2 * QUOTA
    fbase = (m % 2) * nslots
    cbase = ((m + 1) % 2) * nslots

    def mk(e, ci, j, w_hbm, base):
        src = w_hbm.at[e, pl.ds(ci * wc, wc), :]
        return pltpu.make_async_copy(src, stg_ref.at[base + j],
                                     sem_ref.at[base + j])

    srcs = [w1_ref, w1_ref, w3_ref, w3_ref]

    @pl.when(m == 0)
    def _prime():
        for j in range(nslots):
            mk(cen_ref[m], cidx_ref[m, j % QUOTA], j, srcs[j], cbase).start()

    for j in range(nslots):
        mk(en_ref[m], fidx_ref[m, j % QUOTA], j, srcs[j], fbase).start()

    xb = xg_ref[...].astype(jnp.bfloat16)
    dn = (((1,), (1,)), ((), ()))
    x1 = lax.dot_general(xb, w1b_ref[par], dn,
                         preferred_element_type=jnp.float32)
    x3 = lax.dot_general(xb, w3b_ref[par], dn,
                         preferred_element_type=jnp.float32)
    h_ref[...] = (x1 * jax.nn.sigmoid(x1) * x3).astype(jnp.bfloat16)

    dsts = [w1b_ref, w1b_ref, w3b_ref, w3b_ref]
    cslab = cslab_ref[m]
    for j in range(nslots):
        ci = cidx_ref[m, j % QUOTA]
        mk(cen_ref[m], ci, j, srcs[j], cbase).wait()
        dsts[j][cslab, pl.ds(ci * wc, wc), :] = (
            stg_ref[cbase + j].astype(jnp.bfloat16))

    @pl.when(m == pl.num_programs(0) - 1)
    def _drain():
        for j in range(nslots):
            mk(en_ref[m], fidx_ref[m, j % QUOTA], j, srcs[j], fbase).wait()


def _stage2_body(par_ref, en_ref, fidx_ref, cen_ref, cidx_ref, cslab_ref,
                 h_ref, w2_ref, out_ref, w2b_ref, stg_ref, sem_ref):
    m = pl.program_id(0)
    d = w2b_ref.shape[1]
    wc = d // NCH
    par = par_ref[m]
    fbase = (m % 2) * QUOTA
    cbase = ((m + 1) % 2) * QUOTA

    def mk(e, ci, j, base):
        src = w2_ref.at[e, pl.ds(ci * wc, wc), :]
        return pltpu.make_async_copy(src, stg_ref.at[base + j],
                                     sem_ref.at[base + j])

    @pl.when(m == 0)
    def _prime():
        for j in range(QUOTA):
            mk(cen_ref[m], cidx_ref[m, j], j, cbase).start()

    for j in range(QUOTA):
        mk(en_ref[m], fidx_ref[m, j], j, fbase).start()

    dn = (((1,), (1,)), ((), ()))
    out_ref[...] = lax.dot_general(h_ref[...], w2b_ref[par], dn,
                                   preferred_element_type=jnp.float32)

    cslab = cslab_ref[m]
    for j in range(QUOTA):
        ci = cidx_ref[m, j]
        mk(cen_ref[m], ci, j, cbase).wait()
        w2b_ref[cslab, pl.ds(ci * wc, wc), :] = (
            stg_ref[cbase + j].astype(jnp.bfloat16))

    @pl.when(m == pl.num_programs(0) - 1)
    def _drain():
        for j in range(QUOTA):
            mk(en_ref[m], fidx_ref[m, j], j, fbase).wait()


def _grouped_ffn(xg, w1, w3, w2, par, en, fidx, cen, cidx, cslab, ngrid,
                 cap_rows):
    num_e, inter, dim = w1.shape
    wc13 = inter // NCH
    wc2 = dim // NCH

    h = pl.pallas_call(
        _stage13_body,
        grid_spec=pltpu.PrefetchScalarGridSpec(
            num_scalar_prefetch=6,
            grid=(ngrid,),
            in_specs=[
                pl.BlockSpec((BM, dim), lambda m, *_: (m, 0)),
                pl.BlockSpec(memory_space=pltpu.MemorySpace.HBM),
                pl.BlockSpec(memory_space=pltpu.MemorySpace.HBM),
            ],
            out_specs=pl.BlockSpec((BM, inter), lambda m, *_: (m, 0)),
            scratch_shapes=[
                pltpu.VMEM((2, inter, dim), jnp.bfloat16),
                pltpu.VMEM((2, inter, dim), jnp.bfloat16),
                pltpu.VMEM((2 * 2 * QUOTA, wc13, dim), jnp.float32),
                pltpu.SemaphoreType.DMA((2 * 2 * QUOTA,)),
            ],
        ),
        out_shape=jax.ShapeDtypeStruct((cap_rows, inter), jnp.bfloat16),
        compiler_params=pltpu.CompilerParams(
            dimension_semantics=("arbitrary",)),
    )(par, en, fidx, cen, cidx, cslab, xg, w1, w3)

    y = pl.pallas_call(
        _stage2_body,
        grid_spec=pltpu.PrefetchScalarGridSpec(
            num_scalar_prefetch=6,
            grid=(ngrid,),
            in_specs=[
                pl.BlockSpec((BM, inter), lambda m, *_: (m, 0)),
                pl.BlockSpec(memory_space=pltpu.MemorySpace.HBM),
            ],
            out_specs=pl.BlockSpec((BM, dim), lambda m, *_: (m, 0)),
            scratch_shapes=[
                pltpu.VMEM((2, dim, inter), jnp.bfloat16),
                pltpu.VMEM((2 * QUOTA, wc2, inter), jnp.float32),
                pltpu.SemaphoreType.DMA((2 * QUOTA,)),
            ],
        ),
        out_shape=jax.ShapeDtypeStruct((cap_rows, dim), jnp.float32),
        compiler_params=pltpu.CompilerParams(
            dimension_semantics=("arbitrary",)),
    )(par, en, fidx, cen, cidx, cslab, h, w2)
    return y


def kernel(x, expert_indices, w1, w2, w3):
    seq_len, dim = x.shape
    top_k = expert_indices.shape[1]
    num_e = w1.shape[0]
    p = seq_len * top_k
    m_max = DUMMY + p // BM + (num_e - 1) * MIN_BLKS
    cap_rows = m_max * BM

    e_flat = expert_indices.reshape(-1).astype(jnp.int32)
    oh = (e_flat[:, None] == jnp.arange(num_e, dtype=jnp.int32)[None, :])
    oh = oh.astype(jnp.int32)
    counts = oh.sum(0)
    present = (counts > 0).astype(jnp.int32)
    nb = jnp.maximum((counts + BM - 1) // BM, MIN_BLKS) * present
    starts_blk = DUMMY + jnp.concatenate(
        [jnp.zeros((1,), jnp.int32), jnp.cumsum(nb)[:-1].astype(jnp.int32)])
    rank = (jnp.cumsum(oh, axis=0) * oh).sum(1) - 1
    pos = starts_blk[e_flat] * BM + rank
    ngrid = DUMMY + nb.sum()

    bids = jnp.arange(m_max, dtype=jnp.int32)
    is_dummy = bids < DUMMY
    eb = jnp.clip(
        jnp.searchsorted(starts_blk, bids, side="right").astype(jnp.int32) - 1,
        0, num_e - 1)
    eb = jnp.where(is_dummy, 0, eb)
    runidx = jnp.where(is_dummy, 0, jnp.cumsum(present)[eb])
    par = (runidx % 2).astype(jnp.int32)
    sir = jnp.where(is_dummy, bids, bids - starts_blk[eb])
    rlen = jnp.where(is_dummy, DUMMY, nb[eb])
    big = jnp.where(present > 0, jnp.arange(num_e, dtype=jnp.int32), num_e)
    sufmin = jnp.flip(lax.cummin(jnp.flip(big)))
    nxt = jnp.concatenate([sufmin[1:],
                           jnp.full((1,), num_e, jnp.int32)]).astype(jnp.int32)
    en = jnp.where(is_dummy, sufmin[0], nxt[eb])
    en = jnp.where(en >= num_e, eb, en).astype(jnp.int32)
    slot = sir[:, None] * QUOTA + jnp.arange(QUOTA, dtype=jnp.int32)[None, :]
    fidx = jnp.minimum(slot * NCH // (QUOTA * jnp.maximum(rlen - 1, 1))
                       [:, None], NCH - 1).astype(jnp.int32)
    prev = jnp.maximum(bids - 1, 0)
    cen = en[prev]
    cidx = fidx[prev]
    cslab = (1 - par[prev]).astype(jnp.int32)
    pos_k = pos.reshape(seq_len, top_k).T

    xg = _sc_scatter_rows(x, pos_k, cap_rows)

    y = _grouped_ffn(xg, w1, w3, w2, par, en, fidx, cen, cidx, cslab, ngrid,
                     cap_rows)

    out = _sc_gather_rows(y, pos, 64)
    return out.reshape(seq_len, top_k, dim)
